# baseline (speedup 1.0000x reference)
import functools

import jax
import jax.numpy as jnp
from jax import lax
from jax.experimental import pallas as pl
from jax.experimental.pallas import tpu as pltpu
from jax.experimental.pallas import tpu_sc as plsc

N = 10000
F = 128
H = F // 2
B = 16
NC = 2
NS = 16
CH = 128
NSLOT = 4
K = 164
KP = K + NSLOT
NPAD = 10240
RPT = NPAD // NS
RB = 2000
GRID = N // RB

_mesh = plsc.VectorSubcoreMesh(core_axis_name="c", subcore_axis_name="s")


@functools.partial(
    pl.kernel,
    out_type=jax.ShapeDtypeStruct((NC, NPAD, 16), jnp.float32),
    mesh=_mesh,
    scratch_types=[
        pltpu.VMEM((KP, CH), jnp.int32),
        pltpu.VMEM((CH, 16), jnp.float32),
        pltpu.VMEM_SHARED((NPAD, 16), jnp.float32),
    ],
)
def _sc_degree(dsts_hbm, ones_hbm, z16_hbm, out_hbm, dst_v, ones_v, acc):
    c = lax.axis_index("c")
    s = lax.axis_index("s")
    pltpu.sync_copy(z16_hbm, acc.at[pl.ds(s * RPT, RPT)])
    pltpu.sync_copy(dsts_hbm.at[s], dst_v)
    pltpu.sync_copy(ones_hbm, ones_v)
    plsc.subcore_barrier()

    def body(k, carry):
        pltpu.sync_copy(ones_v, acc.at[dst_v.at[k]], add=True)
        return carry

    lax.fori_loop(c * (K // 2), c * (K // 2) + K // 2, body, 0)
    plsc.subcore_barrier()
    pltpu.sync_copy(acc.at[pl.ds(s * RPT, RPT)],
                    out_hbm.at[c, pl.ds(s * RPT, RPT)])


RST = N // NS


@functools.partial(
    pl.kernel,
    out_type=jax.ShapeDtypeStruct((NC, NPAD, H), jnp.float32),
    mesh=_mesh,
    scratch_types=[
        [pltpu.VMEM((2, CH), jnp.int32)] * NSLOT,
        [pltpu.VMEM((CH, H), jnp.float32)] * NSLOT,
        pltpu.VMEM_SHARED((NPAD, H), jnp.float32),
        pltpu.VMEM_SHARED((N, H), jnp.float32),
        [pltpu.SemaphoreType.DMA] * NSLOT,
        [pltpu.SemaphoreType.DMA] * NSLOT,
        [pltpu.SemaphoreType.DMA] * NSLOT,
    ],
    compiler_params=pltpu.CompilerParams(use_tc_tiling_on_sc=False),
)
def _sc_scatter(g_lo_hbm, g_hi_hbm, sd_hbm, z64_hbm, out_hbm,
                idxs, bufs, acc, gstage, gsems, ssems, isems):
    c = lax.axis_index("c")
    s = lax.axis_index("s")
    pltpu.sync_copy(z64_hbm, acc.at[pl.ds(s * RPT, RPT)])
    pl.when(c == 0)(lambda: pltpu.sync_copy(
        g_lo_hbm.at[pl.ds(s * RST, RST)], gstage.at[pl.ds(s * RST, RST)]))
    pl.when(c == 1)(lambda: pltpu.sync_copy(
        g_hi_hbm.at[pl.ds(s * RST, RST)], gstage.at[pl.ds(s * RST, RST)]))
    for b in range(NSLOT):
        pltpu.sync_copy(sd_hbm.at[s, b], idxs[b])
    plsc.subcore_barrier()

    for b in range(NSLOT):
        pltpu.async_copy(gstage.at[idxs[b].at[0]], bufs[b], gsems[b])

    def body(i, carry):
        g = NSLOT * i
        for b in range(NSLOT):
            pltpu.make_async_copy(gstage.at[idxs[b].at[0]], bufs[b],
                                  gsems[b]).wait()
            pltpu.async_copy(bufs[b], acc.at[idxs[b].at[1]], ssems[b],
                             add=True)
        for b in range(NSLOT):
            k = g + b
            pltpu.make_async_copy(bufs[b], acc.at[idxs[b].at[1]],
                                  ssems[b]).wait()
            pltpu.async_copy(sd_hbm.at[s, k + NSLOT], idxs[b], isems[b])
        for b in range(NSLOT):
            k = g + b
            pltpu.make_async_copy(sd_hbm.at[s, k + NSLOT], idxs[b],
                                  isems[b]).wait()
            pltpu.async_copy(gstage.at[idxs[b].at[0]], bufs[b], gsems[b])
        return carry

    lax.fori_loop(0, K // NSLOT, body, 0)
    for b in range(NSLOT):
        pltpu.make_async_copy(gstage.at[idxs[b].at[0]], bufs[b],
                              gsems[b]).wait()
    plsc.subcore_barrier()
    pltpu.sync_copy(acc.at[pl.ds(s * RPT, RPT)],
                    out_hbm.at[c, pl.ds(s * RPT, RPT)])


def _dinv(dp):
    return lax.rsqrt(dp[0, :, 0:1] + dp[1, :, 0:1])


def _tc_first_body(x_ref, w_ref, dp_ref, glo_ref, ghi_ref):
    g = jnp.dot(x_ref[...], w_ref[...], preferred_element_type=jnp.float32)
    g = _dinv(dp_ref[...]) * g
    glo_ref[...] = g[:, :H]
    ghi_ref[...] = g[:, H:]


def _tc_mid_body(ap_ref, dp_ref, b_ref, w_ref, glo_ref, ghi_ref):
    dinv = _dinv(dp_ref[...])
    a = jnp.concatenate([ap_ref[0], ap_ref[1]], axis=1)
    act = jax.nn.relu(dinv * a + b_ref[...])
    g = dinv * jnp.dot(act, w_ref[...], preferred_element_type=jnp.float32)
    glo_ref[...] = g[:, :H]
    ghi_ref[...] = g[:, H:]


def _tc_final_body(ap_ref, dp_ref, b_ref, oh_ref, xs_ref,
                   wl1a_ref, wl1b_ref, bl1_ref, wl2_ref, bl2_ref,
                   out_ref, macc):
    i = pl.program_id(0)

    @pl.when(i == 0)
    def _():
        macc[...] = jnp.zeros_like(macc)

    dinv = _dinv(dp_ref[...])
    a = jnp.concatenate([ap_ref[0], ap_ref[1]], axis=1)
    act = jax.nn.relu(dinv * a + b_ref[...])
    aug = jnp.concatenate([act, jnp.ones((RB, 8), jnp.float32)], axis=1)
    macc[...] += lax.dot_general(oh_ref[...], aug, (((0,), (0,)), ((), ())),
                                 preferred_element_type=jnp.float32)

    @pl.when(i == GRID - 1)
    def _():
        m = macc[...]
        pooled = m[:, :F] / jnp.maximum(m[:, F:F + 1], 1.0)
        h1 = jax.nn.relu(
            jnp.dot(pooled, wl1a_ref[...], preferred_element_type=jnp.float32)
            + jnp.dot(xs_ref[...], wl1b_ref[...],
                      preferred_element_type=jnp.float32)
            + bl1_ref[...])
        out_ref[...] = (jnp.dot(h1, wl2_ref[...],
                                preferred_element_type=jnp.float32)
                        + bl2_ref[...])


def _half_spec():
    return pl.BlockSpec((RB, H), lambda i: (i, 0))


def _ap_spec():
    return pl.BlockSpec((NC, RB, H), lambda i: (0, i, 0))


def _deg_spec():
    return pl.BlockSpec((NC, RB, 16), lambda i: (0, i, 0))


def _full(shape):
    return pl.BlockSpec(shape, lambda i: tuple(0 for _ in shape))


_g_out = lambda: (
    [jax.ShapeDtypeStruct((N, H), jnp.float32)] * 2,
    [_half_spec(), _half_spec()],
)


def _tc_first(x, w0, degp):
    shapes, specs = _g_out()
    return pl.pallas_call(
        _tc_first_body,
        grid=(GRID,),
        in_specs=[pl.BlockSpec((RB, F), lambda i: (i, 0)),
                  _full((F, F)), _deg_spec()],
        out_specs=specs,
        out_shape=shapes,
    )(x, w0, degp)


def _tc_mid(ap, degp, b, w):
    shapes, specs = _g_out()
    return pl.pallas_call(
        _tc_mid_body,
        grid=(GRID,),
        in_specs=[_ap_spec(), _deg_spec(), _full((1, F)), _full((F, F))],
        out_specs=specs,
        out_shape=shapes,
    )(ap, degp, b, w)


def _tc_final(ap, degp, b, onehot, xs, wl1a, wl1b, bl1, wl2, bl2):
    return pl.pallas_call(
        _tc_final_body,
        grid=(GRID,),
        in_specs=[_ap_spec(), _deg_spec(), _full((1, F)),
                  pl.BlockSpec((RB, B), lambda i: (i, 0)), _full((B, 4)),
                  _full((F, 64)), _full((4, 64)), _full((1, 64)),
                  _full((64, 1)), _full((1, 1))],
        out_specs=_full((B, 1)),
        out_shape=jax.ShapeDtypeStruct((B, 1), jnp.float32),
        scratch_shapes=[pltpu.VMEM((B, F + 8), jnp.float32)],
    )(ap, degp, b, onehot, xs, wl1a, wl1b, bl1, wl2, bl2)


def kernel(x, edge_index, x_scalar, batch_index,
           W0, b0, W1, b1, W2, b2, W3, b3, Wl1, bl1, Wl2, bl2):
    loop = jnp.arange(N, dtype=edge_index.dtype)
    src = jnp.concatenate([edge_index[0], loop])
    dst = jnp.concatenate([edge_index[1], loop])
    ep = NS * K * CH
    pad = ep - src.shape[0]
    pad_src = (jnp.arange(pad, dtype=src.dtype) * 37) % N
    pad_dst = N + (jnp.arange(pad, dtype=dst.dtype) % (NPAD - N))
    src = jnp.concatenate([src, pad_src])
    dst = jnp.concatenate([dst, pad_dst])
    srcs = src.reshape(NS, K, CH)
    dsts = dst.reshape(NS, K, CH)
    npr = (KP - K) * CH
    zrow = ((jnp.arange(npr, dtype=jnp.int32) * 53) % N).reshape(1, KP - K,
                                                                 CH)
    zrow = jnp.broadcast_to(zrow, (NS, KP - K, CH))
    srcs = jnp.concatenate([srcs, zrow], axis=1)
    dsts = jnp.concatenate([dsts, jnp.full((NS, KP - K, CH), N, jnp.int32)],
                           axis=1)
    sd = jnp.stack([srcs, dsts], axis=2)

    ones16 = jnp.ones((CH, 16), jnp.float32)
    z16 = jnp.zeros((RPT, 16), jnp.float32)
    z64 = jnp.zeros((RPT, H), jnp.float32)
    onehot = (batch_index[:, None] ==
              jnp.arange(B, dtype=batch_index.dtype)[None, :]
              ).astype(jnp.float32)

    degp = _sc_degree(dsts, ones16, z16)

    g_lo, g_hi = _tc_first(x, W0, degp)
    for b_l, w_next in ((b0, W1), (b1, W2), (b2, W3)):
        ap = _sc_scatter(g_lo, g_hi, sd, z64)
        g_lo, g_hi = _tc_mid(ap, degp, b_l.reshape(1, F), w_next)
    ap = _sc_scatter(g_lo, g_hi, sd, z64)

    return _tc_final(ap, degp, b3.reshape(1, F), onehot, x_scalar,
                     Wl1[:F], Wl1[F:], bl1.reshape(1, 64),
                     Wl2, bl2.reshape(1, 1))


# speedup vs baseline: 1.3890x; 1.3890x over previous
import functools

import jax
import jax.numpy as jnp
from jax import lax
from jax.experimental import pallas as pl
from jax.experimental.pallas import tpu as pltpu
from jax.experimental.pallas import tpu_sc as plsc

N = 10000
F = 128
H = F // 2
B = 16
NC = 2
NS = 16
CH = 120
NSLOT = 6
K = 174
KP = K + NSLOT
NPAD = 10240
RPT = NPAD // NS
RB = 2000
GRID = N // RB

_mesh = plsc.VectorSubcoreMesh(core_axis_name="c", subcore_axis_name="s")


@functools.partial(
    pl.kernel,
    out_type=jax.ShapeDtypeStruct((NC, NPAD, 16), jnp.float32),
    mesh=_mesh,
    scratch_types=[
        pltpu.VMEM((KP, CH), jnp.int32),
        pltpu.VMEM((CH, 16), jnp.float32),
        pltpu.VMEM_SHARED((NPAD, 16), jnp.float32),
    ],
)
def _sc_degree(dsts_hbm, ones_hbm, z16_hbm, out_hbm, dst_v, ones_v, acc):
    c = lax.axis_index("c")
    s = lax.axis_index("s")
    pltpu.sync_copy(z16_hbm, acc.at[pl.ds(s * RPT, RPT)])
    pltpu.sync_copy(dsts_hbm.at[s], dst_v)
    pltpu.sync_copy(ones_hbm, ones_v)
    plsc.subcore_barrier()

    def body(k, carry):
        pltpu.sync_copy(ones_v, acc.at[dst_v.at[k]], add=True)
        return carry

    lax.fori_loop(c * (K // 2), c * (K // 2) + K // 2, body, 0)
    plsc.subcore_barrier()
    pltpu.sync_copy(acc.at[pl.ds(s * RPT, RPT)],
                    out_hbm.at[c, pl.ds(s * RPT, RPT)])


RST = N // NS


@functools.partial(
    pl.kernel,
    out_type=jax.ShapeDtypeStruct((NC, NPAD, H), jnp.float32),
    mesh=_mesh,
    scratch_types=[
        [pltpu.VMEM((2, CH), jnp.int32)] * NSLOT,
        [pltpu.VMEM((CH, H), jnp.float32)] * NSLOT,
        pltpu.VMEM_SHARED((NPAD, H), jnp.float32),
        pltpu.VMEM_SHARED((N, H), jnp.float32),
        [pltpu.SemaphoreType.DMA] * NSLOT,
        [pltpu.SemaphoreType.DMA] * NSLOT,
        [pltpu.SemaphoreType.DMA] * NSLOT,
    ],
    compiler_params=pltpu.CompilerParams(use_tc_tiling_on_sc=False),
)
def _sc_scatter(g_lo_hbm, g_hi_hbm, sd_hbm, z64_hbm, out_hbm,
                idxs, bufs, acc, gstage, gsems, ssems, isems):
    c = lax.axis_index("c")
    s = lax.axis_index("s")
    pltpu.sync_copy(z64_hbm, acc.at[pl.ds(s * RPT, RPT)])
    pl.when(c == 0)(lambda: pltpu.sync_copy(
        g_lo_hbm.at[pl.ds(s * RST, RST)], gstage.at[pl.ds(s * RST, RST)]))
    pl.when(c == 1)(lambda: pltpu.sync_copy(
        g_hi_hbm.at[pl.ds(s * RST, RST)], gstage.at[pl.ds(s * RST, RST)]))
    for b in range(NSLOT):
        pltpu.sync_copy(sd_hbm.at[s, b], idxs[b])
    plsc.subcore_barrier()

    for b in range(NSLOT):
        pltpu.async_copy(gstage.at[idxs[b].at[0]], bufs[b], gsems[b])

    def body(i, carry):
        g = NSLOT * i
        for b in range(NSLOT):
            pltpu.make_async_copy(gstage.at[idxs[b].at[0]], bufs[b],
                                  gsems[b]).wait()
            pltpu.async_copy(bufs[b], acc.at[idxs[b].at[1]], ssems[b],
                             add=True)
        for b in range(NSLOT):
            k = g + b
            pltpu.make_async_copy(bufs[b], acc.at[idxs[b].at[1]],
                                  ssems[b]).wait()
            pltpu.async_copy(sd_hbm.at[s, k + NSLOT], idxs[b], isems[b])
        for b in range(NSLOT):
            k = g + b
            pltpu.make_async_copy(sd_hbm.at[s, k + NSLOT], idxs[b],
                                  isems[b]).wait()
            pltpu.async_copy(gstage.at[idxs[b].at[0]], bufs[b], gsems[b])
        return carry

    lax.fori_loop(0, K // NSLOT, body, 0)
    for b in range(NSLOT):
        pltpu.make_async_copy(gstage.at[idxs[b].at[0]], bufs[b],
                              gsems[b]).wait()
    plsc.subcore_barrier()
    pltpu.sync_copy(acc.at[pl.ds(s * RPT, RPT)],
                    out_hbm.at[c, pl.ds(s * RPT, RPT)])


def _dinv(dp):
    return lax.rsqrt(dp[0, :, 0:1] + dp[1, :, 0:1])


def _tc_first_body(x_ref, w_ref, dp_ref, glo_ref, ghi_ref):
    g = jnp.dot(x_ref[...], w_ref[...], preferred_element_type=jnp.float32)
    g = _dinv(dp_ref[...]) * g
    glo_ref[...] = g[:, :H]
    ghi_ref[...] = g[:, H:]


def _tc_mid_body(ap_ref, dp_ref, b_ref, w_ref, glo_ref, ghi_ref):
    dinv = _dinv(dp_ref[...])
    a = jnp.concatenate([ap_ref[0], ap_ref[1]], axis=1)
    act = jax.nn.relu(dinv * a + b_ref[...])
    g = dinv * jnp.dot(act, w_ref[...], preferred_element_type=jnp.float32)
    glo_ref[...] = g[:, :H]
    ghi_ref[...] = g[:, H:]


def _tc_final_body(ap_ref, dp_ref, b_ref, oh_ref, xs_ref,
                   wl1a_ref, wl1b_ref, bl1_ref, wl2_ref, bl2_ref,
                   out_ref, macc):
    i = pl.program_id(0)

    @pl.when(i == 0)
    def _():
        macc[...] = jnp.zeros_like(macc)

    dinv = _dinv(dp_ref[...])
    a = jnp.concatenate([ap_ref[0], ap_ref[1]], axis=1)
    act = jax.nn.relu(dinv * a + b_ref[...])
    aug = jnp.concatenate([act, jnp.ones((RB, 8), jnp.float32)], axis=1)
    macc[...] += lax.dot_general(oh_ref[...], aug, (((0,), (0,)), ((), ())),
                                 preferred_element_type=jnp.float32)

    @pl.when(i == GRID - 1)
    def _():
        m = macc[...]
        pooled = m[:, :F] / jnp.maximum(m[:, F:F + 1], 1.0)
        h1 = jax.nn.relu(
            jnp.dot(pooled, wl1a_ref[...], preferred_element_type=jnp.float32)
            + jnp.dot(xs_ref[...], wl1b_ref[...],
                      preferred_element_type=jnp.float32)
            + bl1_ref[...])
        out_ref[...] = (jnp.dot(h1, wl2_ref[...],
                                preferred_element_type=jnp.float32)
                        + bl2_ref[...])


def _half_spec():
    return pl.BlockSpec((RB, H), lambda i: (i, 0))


def _ap_spec():
    return pl.BlockSpec((NC, RB, H), lambda i: (0, i, 0))


def _deg_spec():
    return pl.BlockSpec((NC, RB, 16), lambda i: (0, i, 0))


def _full(shape):
    return pl.BlockSpec(shape, lambda i: tuple(0 for _ in shape))


_g_out = lambda: (
    [jax.ShapeDtypeStruct((N, H), jnp.float32)] * 2,
    [_half_spec(), _half_spec()],
)


def _tc_first(x, w0, degp):
    shapes, specs = _g_out()
    return pl.pallas_call(
        _tc_first_body,
        grid=(GRID,),
        in_specs=[pl.BlockSpec((RB, F), lambda i: (i, 0)),
                  _full((F, F)), _deg_spec()],
        out_specs=specs,
        out_shape=shapes,
    )(x, w0, degp)


def _tc_mid(ap, degp, b, w):
    shapes, specs = _g_out()
    return pl.pallas_call(
        _tc_mid_body,
        grid=(GRID,),
        in_specs=[_ap_spec(), _deg_spec(), _full((1, F)), _full((F, F))],
        out_specs=specs,
        out_shape=shapes,
    )(ap, degp, b, w)


def _tc_final(ap, degp, b, onehot, xs, wl1a, wl1b, bl1, wl2, bl2):
    return pl.pallas_call(
        _tc_final_body,
        grid=(GRID,),
        in_specs=[_ap_spec(), _deg_spec(), _full((1, F)),
                  pl.BlockSpec((RB, B), lambda i: (i, 0)), _full((B, 4)),
                  _full((F, 64)), _full((4, 64)), _full((1, 64)),
                  _full((64, 1)), _full((1, 1))],
        out_specs=_full((B, 1)),
        out_shape=jax.ShapeDtypeStruct((B, 1), jnp.float32),
        scratch_shapes=[pltpu.VMEM((B, F + 8), jnp.float32)],
    )(ap, degp, b, onehot, xs, wl1a, wl1b, bl1, wl2, bl2)


def kernel(x, edge_index, x_scalar, batch_index,
           W0, b0, W1, b1, W2, b2, W3, b3, Wl1, bl1, Wl2, bl2):
    loop = jnp.arange(N, dtype=edge_index.dtype)
    src = jnp.concatenate([edge_index[0], loop])
    dst = jnp.concatenate([edge_index[1], loop])
    ep = NS * K * CH
    pad = ep - src.shape[0]
    pad_src = (jnp.arange(pad, dtype=src.dtype) * 37) % N
    pad_dst = N + (jnp.arange(pad, dtype=dst.dtype) % (NPAD - N))
    src = jnp.concatenate([src, pad_src])
    dst = jnp.concatenate([dst, pad_dst])
    srcs = src.reshape(NS, K, CH)
    dsts = dst.reshape(NS, K, CH)
    npr = (KP - K) * CH
    zrow = ((jnp.arange(npr, dtype=jnp.int32) * 53) % N).reshape(1, KP - K,
                                                                 CH)
    zrow = jnp.broadcast_to(zrow, (NS, KP - K, CH))
    srcs = jnp.concatenate([srcs, zrow], axis=1)
    dsts = jnp.concatenate([dsts, jnp.full((NS, KP - K, CH), N, jnp.int32)],
                           axis=1)
    sd = jnp.stack([srcs, dsts], axis=2)

    ones16 = jnp.ones((CH, 16), jnp.float32)
    z16 = jnp.zeros((RPT, 16), jnp.float32)
    z64 = jnp.zeros((RPT, H), jnp.float32)
    onehot = (batch_index[:, None] ==
              jnp.arange(B, dtype=batch_index.dtype)[None, :]
              ).astype(jnp.float32)

    degp = _sc_degree(dsts, ones16, z16)

    g_lo, g_hi = _tc_first(x, W0, degp)
    for b_l, w_next in ((b0, W1), (b1, W2), (b2, W3)):
        ap = _sc_scatter(g_lo, g_hi, sd, z64)
        g_lo, g_hi = _tc_mid(ap, degp, b_l.reshape(1, F), w_next)
    ap = _sc_scatter(g_lo, g_hi, sd, z64)

    return _tc_final(ap, degp, b3.reshape(1, F), onehot, x_scalar,
                     Wl1[:F], Wl1[F:], bl1.reshape(1, 64),
                     Wl2, bl2.reshape(1, 1))
